# baseline (speedup 1.0000x reference)
import functools

import jax
import jax.numpy as jnp
from jax import lax
from jax.experimental import pallas as pl
from jax.experimental.pallas import tpu as pltpu
from jax.experimental.pallas import tpu_sc as plsc

N_TYPES = 4
N_DESC = 16
K_MAX = 8
R_C = 6.0

NW = 32
LANES = 16


def _sc_gather(xs, ys, zs, idx_all, npad, nn):
    apw = npad // NW
    epw = nn * apw

    @functools.partial(
        pl.kernel,
        out_type=jax.ShapeDtypeStruct((nn, npad), jnp.float32),
        mesh=plsc.VectorSubcoreMesh(core_axis_name="c", subcore_axis_name="s"),
        compiler_params=pltpu.CompilerParams(
            needs_layout_passes=False, use_tc_tiling_on_sc=False
        ),
        scratch_types=[
            pltpu.VMEM((npad,), jnp.float32),
            pltpu.VMEM((npad,), jnp.float32),
            pltpu.VMEM((npad,), jnp.float32),
            pltpu.VMEM((apw * nn,), jnp.int32),
            pltpu.VMEM((nn, apw), jnp.float32),
        ],
    )
    def k(xs_h, ys_h, zs_h, nbr_h, r2_h, xv, yv, zv, iv, r2v):
        wid = lax.axis_index("s") * 2 + lax.axis_index("c")
        base = wid * apw
        pltpu.sync_copy(xs_h, xv)
        pltpu.sync_copy(ys_h, yv)
        pltpu.sync_copy(zs_h, zv)
        pltpu.sync_copy(nbr_h.at[pl.ds(base * nn, apw * nn)], iv)
        col = lax.iota(jnp.int32, LANES) * nn

        def chunk_body(c, _):
            a0 = base + c * LANES
            xi = xv[pl.ds(a0, LANES)]
            yi = yv[pl.ds(a0, LANES)]
            zi = zv[pl.ds(a0, LANES)]
            o = c * LANES
            cbase = col + o * nn

            def slot_body(s, _):
                idx = plsc.load_gather(iv, [cbase + s])
                xj = plsc.load_gather(xv, [idx])
                yj = plsc.load_gather(yv, [idx])
                zj = plsc.load_gather(zv, [idx])
                dx = xj - xi
                dy = yj - yi
                dz = zj - zi
                r2 = dx * dx + dy * dy + dz * dz
                tj = plsc.bitcast(xj, jnp.int32) & 3
                r2i = (plsc.bitcast(r2, jnp.int32) & ~3) | tj
                r2v[s, pl.ds(o, LANES)] = plsc.bitcast(r2i, jnp.float32)
                return 0

            lax.fori_loop(0, nn, slot_body, 0)
            return 0

        lax.fori_loop(0, apw // LANES, chunk_body, 0)
        pltpu.sync_copy(r2v, r2_h.at[:, pl.ds(base, apw)])

    return k(xs, ys, zs, idx_all)


def _tc_body(r2_ref, ti_ref, c_ref, out_ref):
    r2 = r2_ref[...]
    tj = lax.bitcast_convert_type(r2, jnp.int32) & 3
    r = jnp.sqrt(r2)
    rr = r * (1.0 / R_C)
    fc = jnp.where(rr < 1.0, 0.5 * jnp.cos(jnp.pi * rr) + 0.5, 0.0)
    hfc = 0.5 * fc
    x = 2.0 * (rr - 1.0) * (rr - 1.0) - 1.0

    cheb = [jnp.ones_like(x), x]
    for _ in range(2, K_MAX):
        cheb.append(2.0 * x * cheb[-1] - cheb[-2])
    fs = [(t + 1.0) * hfc for t in cheb]

    rows = []
    for t in range(N_TYPES):
        m = (tj == t).astype(jnp.float32)
        for f in fs:
            rows.append(jnp.sum(f * m, axis=0, keepdims=True))
    p = jnp.concatenate(rows, axis=0)

    ti = ti_ref[...]
    acc = None
    for t in range(N_TYPES):
        ct = c_ref[t * N_DESC:(t + 1) * N_DESC, :]
        gt = lax.dot_general(ct, p, (((1,), (0,)), ((), ())),
                             preferred_element_type=jnp.float32)
        gt = jnp.where(ti == t, gt, 0.0)
        acc = gt if acc is None else acc + gt
    out_ref[...] = acc


def kernel(types, positions, radial_neighbors, c_table):
    n, nn = radial_neighbors.shape
    apw = -(-n // (NW * LANES)) * LANES
    npad = NW * apw
    epw = nn * apw

    ts = jnp.pad(types.astype(jnp.int32), (0, npad - n))
    xs = jnp.pad(positions[:, 0], (0, npad - n))
    xs = lax.bitcast_convert_type(
        (lax.bitcast_convert_type(xs, jnp.int32) & ~3) | ts, jnp.float32)
    ys = jnp.pad(positions[:, 1], (0, npad - n))
    zs = jnp.pad(positions[:, 2], (0, npad - n))
    nbr = jnp.pad(radial_neighbors.astype(jnp.int32), ((0, npad - n), (0, 0)))
    idx_all = nbr.reshape(npad * nn)

    r2 = _sc_gather(xs, ys, zs, idx_all, npad, nn)
    ti2d = ts.reshape(1, npad)
    c2 = c_table.transpose(0, 2, 1, 3).reshape(N_TYPES * N_DESC, N_TYPES * K_MAX)

    bn = 512
    grid = (npad // bn,)
    g = pl.pallas_call(
        _tc_body,
        grid=grid,
        in_specs=[
            pl.BlockSpec((nn, bn), lambda j: (0, j)),
            pl.BlockSpec((1, bn), lambda j: (0, j)),
            pl.BlockSpec((N_TYPES * N_DESC, N_TYPES * K_MAX), lambda j: (0, 0)),
        ],
        out_specs=pl.BlockSpec((N_DESC, bn), lambda j: (0, j)),
        out_shape=jax.ShapeDtypeStruct((N_DESC, npad), jnp.float32),
    )(r2, ti2d, c2)

    return g[:, :n].T


# speedup vs baseline: 1.5961x; 1.5961x over previous
import functools

import jax
import jax.numpy as jnp
from jax import lax
from jax.experimental import pallas as pl
from jax.experimental.pallas import tpu as pltpu
from jax.experimental.pallas import tpu_sc as plsc

N_TYPES = 4
N_DESC = 16
K_MAX = 8
R_C = 6.0

NW = 32
LANES = 16


def _sc_gather(xs, ys, zs, idx_all, npad, nn):
    apw = npad // NW
    epw = nn * apw

    @functools.partial(
        pl.kernel,
        out_type=jax.ShapeDtypeStruct((nn, npad), jnp.float32),
        mesh=plsc.VectorSubcoreMesh(core_axis_name="c", subcore_axis_name="s"),
        compiler_params=pltpu.CompilerParams(
            needs_layout_passes=False, use_tc_tiling_on_sc=False
        ),
        scratch_types=[
            pltpu.VMEM((npad,), jnp.float32),
            pltpu.VMEM((npad,), jnp.float32),
            pltpu.VMEM((npad,), jnp.float32),
            pltpu.VMEM((nn, apw), jnp.int32),
            pltpu.VMEM((nn, apw), jnp.float32),
        ],
    )
    def k(xs_h, ys_h, zs_h, nbr_h, r2_h, xv, yv, zv, iv, r2v):
        wid = lax.axis_index("s") * 2 + lax.axis_index("c")
        base = wid * apw
        pltpu.sync_copy(xs_h, xv)
        pltpu.sync_copy(ys_h, yv)
        pltpu.sync_copy(zs_h, zv)
        pltpu.sync_copy(nbr_h.at[:, pl.ds(base, apw)], iv)

        @functools.partial(plsc.parallel_loop, 0, apw // LANES)
        def chunk_body(c):
            a0 = base + c * LANES
            xi = xv[pl.ds(a0, LANES)]
            yi = yv[pl.ds(a0, LANES)]
            zi = zv[pl.ds(a0, LANES)]
            o = c * LANES

            @functools.partial(plsc.parallel_loop, 0, nn, unroll=4)
            def slot_body(s):
                idx = iv[s, pl.ds(o, LANES)]
                xj = plsc.load_gather(xv, [idx])
                yj = plsc.load_gather(yv, [idx])
                zj = plsc.load_gather(zv, [idx])
                dx = xj - xi
                dy = yj - yi
                dz = zj - zi
                r2 = dx * dx + dy * dy + dz * dz
                tj = plsc.bitcast(xj, jnp.int32) & 3
                r2i = (plsc.bitcast(r2, jnp.int32) & ~3) | tj
                r2v[s, pl.ds(o, LANES)] = plsc.bitcast(r2i, jnp.float32)

        pltpu.sync_copy(r2v, r2_h.at[:, pl.ds(base, apw)])

    return k(xs, ys, zs, idx_all)


def _tc_body(r2_ref, ti_ref, c_ref, out_ref):
    r2 = r2_ref[...]
    tj = lax.bitcast_convert_type(r2, jnp.int32) & 3
    r = jnp.sqrt(r2)
    rr = r * (1.0 / R_C)
    fc = jnp.where(rr < 1.0, 0.5 * jnp.cos(jnp.pi * rr) + 0.5, 0.0)
    hfc = 0.5 * fc
    x = 2.0 * (rr - 1.0) * (rr - 1.0) - 1.0

    cheb = [jnp.ones_like(x), x]
    for _ in range(2, K_MAX):
        cheb.append(2.0 * x * cheb[-1] - cheb[-2])
    fs = [(t + 1.0) * hfc for t in cheb]

    rows = []
    for t in range(N_TYPES):
        m = (tj == t).astype(jnp.float32)
        for f in fs:
            rows.append(jnp.sum(f * m, axis=0, keepdims=True))
    p = jnp.concatenate(rows, axis=0)

    ti = ti_ref[...]
    acc = None
    for t in range(N_TYPES):
        ct = c_ref[t * N_DESC:(t + 1) * N_DESC, :]
        gt = lax.dot_general(ct, p, (((1,), (0,)), ((), ())),
                             preferred_element_type=jnp.float32)
        gt = jnp.where(ti == t, gt, 0.0)
        acc = gt if acc is None else acc + gt
    out_ref[...] = acc


def kernel(types, positions, radial_neighbors, c_table):
    n, nn = radial_neighbors.shape
    apw = -(-n // (NW * LANES)) * LANES
    npad = NW * apw
    epw = nn * apw

    ts = jnp.pad(types.astype(jnp.int32), (0, npad - n))
    xs = jnp.pad(positions[:, 0], (0, npad - n))
    xs = lax.bitcast_convert_type(
        (lax.bitcast_convert_type(xs, jnp.int32) & ~3) | ts, jnp.float32)
    ys = jnp.pad(positions[:, 1], (0, npad - n))
    zs = jnp.pad(positions[:, 2], (0, npad - n))
    nbr = jnp.pad(radial_neighbors.astype(jnp.int32), ((0, npad - n), (0, 0)))
    idx_all = nbr.T

    r2 = _sc_gather(xs, ys, zs, idx_all, npad, nn)
    ti2d = ts.reshape(1, npad)
    c2 = c_table.transpose(0, 2, 1, 3).reshape(N_TYPES * N_DESC, N_TYPES * K_MAX)

    bn = 512
    grid = (npad // bn,)
    g = pl.pallas_call(
        _tc_body,
        grid=grid,
        in_specs=[
            pl.BlockSpec((nn, bn), lambda j: (0, j)),
            pl.BlockSpec((1, bn), lambda j: (0, j)),
            pl.BlockSpec((N_TYPES * N_DESC, N_TYPES * K_MAX), lambda j: (0, 0)),
        ],
        out_specs=pl.BlockSpec((N_DESC, bn), lambda j: (0, j)),
        out_shape=jax.ShapeDtypeStruct((N_DESC, npad), jnp.float32),
    )(r2, ti2d, c2)

    return g[:, :n].T
